# Initial kernel scaffold; baseline (speedup 1.0000x reference)
import jax
import jax.numpy as jnp
from jax.experimental import pallas as pl


def kernel(point_clouds, sigma, alphas):
    raise NotImplementedError("write your pallas kernel here")


# speedup vs baseline: 44.7523x; 44.7523x over previous
import jax
import jax.numpy as jnp
from jax import lax
from jax.experimental import pallas as pl

_J = 3
_NW = 4
_D = 3


def _body(pc_ref, aexp_ref, out_ref):
    n = pc_ref.shape[0]
    m = jnp.sum(pc_ref[...], axis=0, keepdims=True) * (1.0 / n)
    s_idx = lax.broadcasted_iota(jnp.int32, (6, 72), 0)
    c_idx = lax.broadcasted_iota(jnp.int32, (6, 72), 1)
    sel = jnp.where(s_idx == 3 * (c_idx // 36) + c_idx % 3, 1.0, 0.0).astype(
        jnp.float32
    )
    m_exp = jnp.dot(m, sel, preferred_element_type=jnp.float32)
    out_ref[...] = m_exp * aexp_ref[...]


def kernel(point_clouds, sigma, alphas):
    del sigma
    b_pc, n, d = point_clouds.shape
    pc2 = point_clouds.transpose(1, 0, 2).reshape(n, b_pc * d)
    a36 = jnp.tile(alphas[:, None, :], (1, _J, 1)).reshape(1, _NW * _J * _D)
    aexp = jnp.tile(a36, (1, b_pc))
    out = pl.pallas_call(
        _body,
        out_shape=jax.ShapeDtypeStruct((1, b_pc * _NW * _J * _D), jnp.float32),
    )(pc2, aexp)
    return out.reshape(b_pc, _NW * _J * _D)
